# baseline (speedup 1.0000x reference)
import functools

import jax
import jax.numpy as jnp
from jax import lax
from jax.experimental import pallas as pl
from jax.experimental.pallas import tpu as pltpu
from jax.experimental.pallas import tpu_sc as plsc

_VOCAB = 100000
_DIM = 64
_BATCH = 16384
_NC = 2
_NS = 16
_L = 16
_NW = _NC * _NS
_BPW = _BATCH // _NW
_GROUPS = _BPW // _L


def _sc_body(s_hbm, d_hbm, head_hbm, tail_hbm, out_hbm,
             s_v, d_v, hrows_v, trows_v, out_v, sem_h, sem_t):
    wid = lax.axis_index("s") * _NC + lax.axis_index("c")
    base = wid * _BPW
    pltpu.sync_copy(s_hbm.at[pl.ds(base, _BPW)], s_v)
    pltpu.sync_copy(d_hbm.at[pl.ds(base, _BPW)], d_v)
    ch = pltpu.async_copy(head_hbm.at[s_v], hrows_v, sem_h)
    ct = pltpu.async_copy(tail_hbm.at[d_v], trows_v, sem_t)
    ch.wait()
    ct.wait()

    def group_step(g, carry):
        lanes = lax.iota(jnp.int32, _L) + g * _L

        def dim_step(j, acc):
            jv = jnp.full((_L,), j, jnp.int32)
            h = plsc.load_gather(hrows_v, [lanes, jv])
            t = plsc.load_gather(trows_v, [lanes, jv])
            return acc + h * t

        acc = lax.fori_loop(0, _DIM, dim_step, jnp.zeros((_L,), jnp.float32))
        out_v[pl.ds(g * _L, _L)] = 1.0 / (1.0 + jnp.exp(acc))
        return carry

    lax.fori_loop(0, _GROUPS, group_step, 0)
    pltpu.sync_copy(out_v, out_hbm.at[pl.ds(base, _BPW)])


_sc_kernel = functools.partial(
    pl.kernel,
    out_type=jax.ShapeDtypeStruct((_BATCH,), jnp.float32),
    mesh=plsc.VectorSubcoreMesh(core_axis_name="c", subcore_axis_name="s",
                                num_cores=_NC, num_subcores=_NS),
    compiler_params=pltpu.CompilerParams(needs_layout_passes=False,
                                         use_tc_tiling_on_sc=False),
    scratch_types=[
        pltpu.VMEM((_BPW,), jnp.int32),
        pltpu.VMEM((_BPW,), jnp.int32),
        pltpu.VMEM((_BPW, _DIM), jnp.float32),
        pltpu.VMEM((_BPW, _DIM), jnp.float32),
        pltpu.VMEM((_BPW,), jnp.float32),
        pltpu.SemaphoreType.DMA,
        pltpu.SemaphoreType.DMA,
    ],
)(_sc_body)


def kernel(x, emb_head, emb_tail):
    s = x[:, 0]
    d = x[:, 1]
    return _sc_kernel(s, d, emb_head, emb_tail)


# speedup vs baseline: 1.0015x; 1.0015x over previous
import functools

import jax
import jax.numpy as jnp
from jax import lax
from jax.experimental import pallas as pl
from jax.experimental.pallas import tpu as pltpu
from jax.experimental.pallas import tpu_sc as plsc

_VOCAB = 100000
_DIM = 64
_BATCH = 16384
_NC = 2
_NS = 16
_L = 16
_NW = _NC * _NS
_BPW = _BATCH // _NW
_GROUPS = _BPW // _L


def _sc_body(s_hbm, d_hbm, head_hbm, tail_hbm, out_hbm,
             s_v, d_v, hrows_v, trows_v, out_v, sem_h, sem_t):
    wid = lax.axis_index("s") * _NC + lax.axis_index("c")
    base = wid * _BPW
    pltpu.sync_copy(s_hbm.at[pl.ds(base, _BPW)], s_v)
    pltpu.sync_copy(d_hbm.at[pl.ds(base, _BPW)], d_v)
    ch = pltpu.async_copy(head_hbm.at[s_v], hrows_v, sem_h)
    ct = pltpu.async_copy(tail_hbm.at[d_v], trows_v, sem_t)
    ch.wait()
    ct.wait()

    def group_step(g, carry):
        lanes = lax.iota(jnp.int32, _L) + g * _L
        acc = jnp.zeros((_L,), jnp.float32)
        for j in range(_DIM):
            jv = jnp.full((_L,), j, jnp.int32)
            h = plsc.load_gather(hrows_v, [lanes, jv])
            t = plsc.load_gather(trows_v, [lanes, jv])
            acc = acc + h * t
        out_v[pl.ds(g * _L, _L)] = 1.0 / (1.0 + jnp.exp(acc))
        return carry

    lax.fori_loop(0, _GROUPS, group_step, 0)
    pltpu.sync_copy(out_v, out_hbm.at[pl.ds(base, _BPW)])


_sc_kernel = functools.partial(
    pl.kernel,
    out_type=jax.ShapeDtypeStruct((_BATCH,), jnp.float32),
    mesh=plsc.VectorSubcoreMesh(core_axis_name="c", subcore_axis_name="s",
                                num_cores=_NC, num_subcores=_NS),
    compiler_params=pltpu.CompilerParams(needs_layout_passes=False,
                                         use_tc_tiling_on_sc=False),
    scratch_types=[
        pltpu.VMEM((_BPW,), jnp.int32),
        pltpu.VMEM((_BPW,), jnp.int32),
        pltpu.VMEM((_BPW, _DIM), jnp.float32),
        pltpu.VMEM((_BPW, _DIM), jnp.float32),
        pltpu.VMEM((_BPW,), jnp.float32),
        pltpu.SemaphoreType.DMA,
        pltpu.SemaphoreType.DMA,
    ],
)(_sc_body)


def kernel(x, emb_head, emb_tail):
    s = x[:, 0]
    d = x[:, 1]
    return _sc_kernel(s, d, emb_head, emb_tail)
